# Initial kernel scaffold; baseline (speedup 1.0000x reference)
import jax
import jax.numpy as jnp
from jax.experimental import pallas as pl


def kernel(nodes, edges, globals_, senders, receivers, params):
    raise NotImplementedError("write your pallas kernel here")


# speedup vs baseline: 4.9160x; 4.9160x over previous
import functools

import jax
import jax.numpy as jnp
from jax import lax
from jax.experimental import pallas as pl
from jax.experimental.pallas import tpu as pltpu
from jax.experimental.pallas import tpu_sc as plsc

_NE = 320000
_NN = 10000
_L = 16
_NER = _NE // 8
_BR = 2000

_C = 80
_PT = _NE // 32
_NCH = _PT // _C
_NNP = 10240
_RT = _NNP // 16


def _mlpw(p):
    return p[0]["W"], p[0]["b"], p[1]["W"], p[1]["b"]


def _enc_edges_body(x, w1, bias1, q, c, h, pre):
    hv = jnp.maximum(x[...] @ w1[...] + bias1[...], 0.0)
    h[...] = hv
    pre[...] = hv @ q[...] + c[...]


def _call_enc_edges(edges_r, w1, bias1, q, c):
    return pl.pallas_call(
        _enc_edges_body,
        grid=(_NER // _BR,),
        in_specs=[
            pl.BlockSpec((_BR, 128), lambda i: (i, 0)),
            pl.BlockSpec((128, 128), lambda i: (0, 0)),
            pl.BlockSpec((1, 128), lambda i: (0, 0)),
            pl.BlockSpec((128, 128), lambda i: (0, 0)),
            pl.BlockSpec((1, 128), lambda i: (0, 0)),
        ],
        out_specs=[pl.BlockSpec((_BR, 128), lambda i: (i, 0)),
                   pl.BlockSpec((_BR, 128), lambda i: (i, 0))],
        out_shape=[jax.ShapeDtypeStruct((_NER, 128), jnp.float32),
                   jax.ShapeDtypeStruct((_NER, 128), jnp.float32)],
    )(edges_r, w1, bias1, q, c)


def _pre_body(hp, he, pk, qk, c, out):
    out[...] = hp[...] @ pk[...] + he[...] @ qk[...] + c[...]


def _call_pre(hp_r, he_r, pk, qk, c):
    return pl.pallas_call(
        _pre_body,
        grid=(_NER // _BR,),
        in_specs=[
            pl.BlockSpec((_BR, 128), lambda i: (i, 0)),
            pl.BlockSpec((_BR, 128), lambda i: (i, 0)),
            pl.BlockSpec((128, 128), lambda i: (0, 0)),
            pl.BlockSpec((128, 128), lambda i: (0, 0)),
            pl.BlockSpec((1, 128), lambda i: (0, 0)),
        ],
        out_specs=pl.BlockSpec((_BR, 128), lambda i: (i, 0)),
        out_shape=jax.ShapeDtypeStruct((_NER, 128), jnp.float32),
    )(hp_r, he_r, pk, qk, c)


def _dec_edges_body(h2, m1, v1, m2, v2, out):
    t = jnp.maximum(h2[...] @ m1[...] + v1[...], 0.0)
    out[...] = t @ m2[...] + v2[...]


def _call_dec_edges(h2_r, m1, v1, m2, v2):
    return pl.pallas_call(
        _dec_edges_body,
        grid=(_NER // _BR,),
        in_specs=[
            pl.BlockSpec((_BR, 128), lambda i: (i, 0)),
            pl.BlockSpec((128, 128), lambda i: (0, 0)),
            pl.BlockSpec((1, 128), lambda i: (0, 0)),
            pl.BlockSpec((128, 8), lambda i: (0, 0)),
            pl.BlockSpec((1, 8), lambda i: (0, 0)),
        ],
        out_specs=pl.BlockSpec((_BR, 8), lambda i: (i, 0)),
        out_shape=jax.ShapeDtypeStruct((_NER, 8), jnp.float32),
    )(h2_r, m1, v1, m2, v2)


def _enc_nodes_body(nodes, glob, wn1, bn1, wn2, bn2, wg1, bg1, wg2, bg2,
                    s0, r0, g0w, cst0, n0, g0, ns0, nr0, c0):
    n = jnp.maximum(nodes[...] @ wn1[...] + bn1[...], 0.0) @ wn2[...] + bn2[...]
    g = jnp.maximum(glob[...] @ wg1[...] + bg1[...], 0.0) @ wg2[...] + bg2[...]
    n0[...] = n
    g0[...] = g
    ns0[...] = n @ s0[...]
    nr0[...] = n @ r0[...]
    c0[...] = g @ g0w[...] + cst0[...]


def _call_enc_nodes(nodes, glob, weights):
    return pl.pallas_call(
        _enc_nodes_body,
        out_shape=[
            jax.ShapeDtypeStruct((_NN, _L), jnp.float32),
            jax.ShapeDtypeStruct((1, _L), jnp.float32),
            jax.ShapeDtypeStruct((_NN, _L), jnp.float32),
            jax.ShapeDtypeStruct((_NN, _L), jnp.float32),
            jax.ShapeDtypeStruct((1, _L), jnp.float32),
        ],
    )(nodes, glob, *weights)


def _stage_common(hsum, cnt, n_in, g_in, w2, b2,
                  wm1, bm1, wm2, bm2, wq1, bq1, wq2, bq2):
    chat = jnp.maximum(cnt, 1.0)
    mask = (cnt > 0.0).astype(jnp.float32)
    agg = (hsum @ w2) / chat + mask * b2
    gb = jnp.broadcast_to(g_in, (_NN, g_in.shape[1]))
    x = jnp.concatenate([agg, n_in, gb], axis=1)
    n_new = jnp.maximum(x @ wm1 + bm1, 0.0) @ wm2 + bm2
    esum = jnp.sum(hsum, axis=0, keepdims=True)
    mean_e = (esum * (1.0 / _NE)) @ w2 + b2
    mean_n = jnp.sum(n_new, axis=0, keepdims=True) * (1.0 / _NN)
    gx = jnp.concatenate([mean_e, mean_n, g_in], axis=1)
    g_new = jnp.maximum(gx @ wq1 + bq1, 0.0) @ wq2 + bq2
    return n_new, g_new


def _stage01_body(is_first, *refs):
    (hsumP, cref, npv, n0, gpv, g0, w2, b2,
     wm1, bm1, wm2, bm2, wq1, bq1, wq2, bq2,
     sa, sb, ra, rb, ga, gb_, cstn,
     n_out, g_out, cnt_out, ns_out, nr_out, c_out) = refs
    hsum = (hsumP[0] + hsumP[1])[0:_NN]
    if is_first:
        cnt = (cref[0] + cref[1])[0:_NN, 0:1]
        n_in = npv[...]
        g_in = gpv[...]
    else:
        cnt = cref[...]
        n_in = jnp.concatenate([npv[...], n0[...]], axis=1)
        g_in = jnp.concatenate([gpv[...], g0[...]], axis=1)
    n_new, g_new = _stage_common(hsum, cnt, n_in, g_in, w2[...], b2[...],
                                 wm1[...], bm1[...], wm2[...], bm2[...],
                                 wq1[...], bq1[...], wq2[...], bq2[...])
    n_out[...] = n_new
    g_out[...] = g_new
    cnt_out[...] = cnt
    ns_out[...] = n_new @ sa[...] + n0[...] @ sb[...]
    nr_out[...] = n_new @ ra[...] + n0[...] @ rb[...]
    c_out[...] = g_new @ ga[...] + g0[...] @ gb_[...] + cstn[...]


def _call_stage01(is_first, args):
    return pl.pallas_call(
        functools.partial(_stage01_body, is_first),
        compiler_params=pltpu.CompilerParams(vmem_limit_bytes=100 * 1024 * 1024),
        out_shape=[
            jax.ShapeDtypeStruct((_NN, _L), jnp.float32),
            jax.ShapeDtypeStruct((1, _L), jnp.float32),
            jax.ShapeDtypeStruct((_NN, 1), jnp.float32),
            jax.ShapeDtypeStruct((_NN, _L), jnp.float32),
            jax.ShapeDtypeStruct((_NN, _L), jnp.float32),
            jax.ShapeDtypeStruct((1, _L), jnp.float32),
        ],
    )(*args)


def _stage2_body(*refs):
    (hsumP, cref, npv, n0, gpv, g0, w2, b2,
     wm1, bm1, wm2, bm2, wq1, bq1, wq2, bq2,
     dn1, dnb1, mn, vn, dg1, dgb1, mg, vg,
     nout, gout) = refs
    hsum = (hsumP[0] + hsumP[1])[0:_NN]
    cnt = cref[...]
    n_in = jnp.concatenate([npv[...], n0[...]], axis=1)
    g_in = jnp.concatenate([gpv[...], g0[...]], axis=1)
    n_new, g_new = _stage_common(hsum, cnt, n_in, g_in, w2[...], b2[...],
                                 wm1[...], bm1[...], wm2[...], bm2[...],
                                 wq1[...], bq1[...], wq2[...], bq2[...])
    tn = jnp.maximum(n_new @ dn1[...] + dnb1[...], 0.0)
    nout[...] = tn @ mn[...] + vn[...]
    tg = jnp.maximum(g_new @ dg1[...] + dgb1[...], 0.0)
    gout[...] = tg @ mg[...] + vg[...]


def _call_stage2(args):
    return pl.pallas_call(
        _stage2_body,
        compiler_params=pltpu.CompilerParams(vmem_limit_bytes=100 * 1024 * 1024),
        out_shape=[
            jax.ShapeDtypeStruct((_NN, 2), jnp.float32),
            jax.ShapeDtypeStruct((1, 1), jnp.float32),
        ],
    )(*args)


def _make_sc_pass(with_count):
    mesh = plsc.VectorSubcoreMesh(core_axis_name="c", subcore_axis_name="s")
    outs = [
        jax.ShapeDtypeStruct((_NE, _L), jnp.float32),
        jax.ShapeDtypeStruct((2, _NNP, _L), jnp.float32),
    ]
    scratch = [
        pltpu.VMEM((_C,), jnp.int32),
        pltpu.VMEM((_C,), jnp.int32),
        pltpu.VMEM((_C, _L), jnp.float32),
        pltpu.VMEM((_C, _L), jnp.float32),
        pltpu.VMEM((_C, _L), jnp.float32),
        pltpu.VMEM((_C, _L), jnp.float32),
        pltpu.VMEM((_RT, _L), jnp.float32),
        pltpu.VMEM_SHARED((_NNP, _L), jnp.float32),
        pltpu.SemaphoreType.DMA,
        pltpu.SemaphoreType.DMA,
    ]
    if with_count:
        outs.append(jax.ShapeDtypeStruct((2, _NNP, _L), jnp.float32))
        scratch += [
            pltpu.VMEM((_C, _L), jnp.float32),
            pltpu.VMEM_SHARED((_NNP, _L), jnp.float32),
        ]

    def body(pre_hbm, snd_hbm, rcv_hbm, ns_hbm, nr_hbm, *rest):
        if with_count:
            (h_hbm, hsum_hbm, cnt_hbm, ids, idr, prev, srow, rrow, hv, zbuf,
             hsum_sh, sem1, sem2, ones_v, cnt_sh) = rest
        else:
            (h_hbm, hsum_hbm, ids, idr, prev, srow, rrow, hv, zbuf,
             hsum_sh, sem1, sem2) = rest
        cid = lax.axis_index("c")
        sid = lax.axis_index("s")
        wid = cid * 16 + sid

        def zrow(i, carry):
            zbuf[i] = jnp.zeros((_L,), jnp.float32)
            return carry
        lax.fori_loop(0, _RT, zrow, 0)
        pltpu.sync_copy(zbuf, hsum_sh.at[pl.ds(sid * _RT, _RT)])
        if with_count:
            pltpu.sync_copy(zbuf, cnt_sh.at[pl.ds(sid * _RT, _RT)])

            def orow(i, carry):
                ones_v[i] = jnp.ones((_L,), jnp.float32)
                return carry
            lax.fori_loop(0, _C, orow, 0)
        plsc.subcore_barrier()

        def chunk(k, carry):
            base = wid * _PT + k * _C
            pltpu.sync_copy(snd_hbm.at[pl.ds(base, _C)], ids)
            pltpu.sync_copy(rcv_hbm.at[pl.ds(base, _C)], idr)
            pltpu.sync_copy(pre_hbm.at[pl.ds(base, _C)], prev)
            cp1 = pltpu.async_copy(ns_hbm.at[ids], srow, sem1)
            cp2 = pltpu.async_copy(nr_hbm.at[idr], rrow, sem2)
            cp1.wait()
            cp2.wait()

            def row(i, c2):
                hv[i] = jnp.maximum(prev[i] + srow[i] + rrow[i], 0.0)
                return c2
            lax.fori_loop(0, _C, row, 0, unroll=8)
            pltpu.sync_copy(hv, h_hbm.at[pl.ds(base, _C)])
            pltpu.sync_copy(hv, hsum_sh.at[idr], add=True)
            if with_count:
                pltpu.sync_copy(ones_v, cnt_sh.at[idr], add=True)
            return carry
        lax.fori_loop(0, _NCH, chunk, 0)

        plsc.subcore_barrier()
        pltpu.sync_copy(hsum_sh.at[pl.ds(sid * _RT, _RT)], zbuf)
        pltpu.sync_copy(zbuf, hsum_hbm.at[cid, pl.ds(sid * _RT, _RT)])
        if with_count:
            pltpu.sync_copy(cnt_sh.at[pl.ds(sid * _RT, _RT)], zbuf)
            pltpu.sync_copy(zbuf, cnt_hbm.at[cid, pl.ds(sid * _RT, _RT)])

    return pl.kernel(body, out_type=outs, scratch_types=scratch, mesh=mesh,
                     compiler_params=pltpu.CompilerParams(
                         use_tc_tiling_on_sc=False))


@functools.lru_cache(maxsize=2)
def _sc_pass_cached(with_count):
    return _make_sc_pass(with_count)


def _sc_pass_count(*args):
    return _sc_pass_cached(True)(*args)


def _sc_pass(*args):
    return _sc_pass_cached(False)(*args)


def kernel(nodes, edges, globals_, senders, receivers, params):
    p = params
    snd = senders.astype(jnp.int32)
    rcv = receivers.astype(jnp.int32)

    We1, be1, We2, be2 = _mlpw(p["enc"]["edge"])
    Wn1, bn1, Wn2, bn2 = _mlpw(p["enc"]["node"])
    Wg1, bg1, Wg2, bg2 = _mlpw(p["enc"]["glob"])

    W1 = [p["core%d" % i]["edge"][0]["W"] for i in range(3)]
    b1 = [p["core%d" % i]["edge"][0]["b"] for i in range(3)]
    W2 = [p["core%d" % i]["edge"][1]["W"] for i in range(3)]
    b2 = [p["core%d" % i]["edge"][1]["b"] for i in range(3)]

    E0, S0, R0, G0 = W1[0][0:16], W1[0][16:32], W1[0][32:48], W1[0][48:64]
    Ea, Eb, Sa, Sb, Ra, Rb, Ga, Gb = ({} for _ in range(8))
    for i in (1, 2):
        w = W1[i]
        Ea[i], Eb[i] = w[0:16], w[16:32]
        Sa[i], Sb[i] = w[32:48], w[48:64]
        Ra[i], Rb[i] = w[64:80], w[80:96]
        Ga[i], Gb[i] = w[96:112], w[112:128]

    eye8 = jnp.eye(8, dtype=jnp.float32)
    kron = lambda w: jnp.kron(eye8, w)
    t8 = lambda v: jnp.tile(v.reshape(1, -1), (1, 8))
    r1 = lambda v: v.reshape(1, -1)

    cst0 = r1(be2 @ E0 + b1[0])
    cstn = {i: r1(b2[i - 1] @ Ea[i] + be2 @ Eb[i] + b1[i]) for i in (1, 2)}

    enc_node_w = (Wn1, r1(bn1), Wn2, r1(bn2), Wg1, r1(bg1), Wg2, r1(bg2),
                  S0, R0, G0, cst0)
    n0, g0, ns0, nr0, c0 = _call_enc_nodes(nodes, globals_, enc_node_w)

    edges_r = edges.reshape(_NER, 128)
    henc_r, pre0_r = _call_enc_edges(edges_r, kron(We1), t8(be1),
                                     kron(We2 @ E0), jnp.tile(c0, (1, 8)))

    def node_mlp_w(i):
        wm1, bm1, wm2, bm2 = _mlpw(p["core%d" % i]["node"])
        wq1, bq1, wq2, bq2 = _mlpw(p["core%d" % i]["glob"])
        return (wm1, r1(bm1), wm2, r1(bm2), wq1, r1(bq1), wq2, r1(bq2))

    h0, hsumP0, cntP0 = _sc_pass_count(pre0_r.reshape(_NE, _L), snd, rcv,
                                       ns0, nr0)
    args0 = (hsumP0, cntP0, n0, n0, g0, g0, W2[0], r1(b2[0]),
             *node_mlp_w(0), Sa[1], Sb[1], Ra[1], Rb[1], Ga[1], Gb[1], cstn[1])
    n1, g1, cnt, ns1, nr1, c1 = _call_stage01(True, args0)

    pre1_r = _call_pre(h0.reshape(_NER, 128), henc_r,
                       kron(W2[0] @ Ea[1]), kron(We2 @ Eb[1]),
                       jnp.tile(c1, (1, 8)))
    h1, hsumP1 = _sc_pass(pre1_r.reshape(_NE, _L), snd, rcv, ns1, nr1)
    args1 = (hsumP1, cnt, n1, n0, g1, g0, W2[1], r1(b2[1]),
             *node_mlp_w(1), Sa[2], Sb[2], Ra[2], Rb[2], Ga[2], Gb[2], cstn[2])
    n2, g2, _, ns2, nr2, c2 = _call_stage01(False, args1)

    pre2_r = _call_pre(h1.reshape(_NER, 128), henc_r,
                       kron(W2[1] @ Ea[2]), kron(We2 @ Eb[2]),
                       jnp.tile(c2, (1, 8)))
    h2, hsumP2 = _sc_pass(pre2_r.reshape(_NE, _L), snd, rcv, ns2, nr2)

    De1, de1, De2, de2 = _mlpw(p["dec"]["edge"])
    Dn1, dn1, Dn2, dn2 = _mlpw(p["dec"]["node"])
    Dg1, dg1, Dg2, dg2 = _mlpw(p["dec"]["glob"])
    Oe, oe = p["out"]["edge"]["W"], p["out"]["edge"]["b"]
    On, on = p["out"]["node"]["W"], p["out"]["node"]["b"]
    Og, og = p["out"]["glob"]["W"], p["out"]["glob"]["b"]

    args2 = (hsumP2, cnt, n2, n0, g2, g0, W2[2], r1(b2[2]), *node_mlp_w(2),
             Dn1, r1(dn1), Dn2 @ On, r1(dn2 @ On + on),
             Dg1, r1(dg1), Dg2 @ Og, r1(dg2 @ Og + og))
    out_n, out_g = _call_stage2(args2)

    m2e = De2 @ Oe
    v2e = (de2 @ Oe + oe)
    oute_r = _call_dec_edges(h2.reshape(_NER, 128),
                             kron(W2[2] @ De1), t8(b2[2] @ De1 + de1),
                             kron(m2e), t8(v2e))
    out_e = oute_r.reshape(_NE, 1)

    return out_n, out_e, out_g


# speedup vs baseline: 6.3934x; 1.3005x over previous
import functools

import jax
import jax.numpy as jnp
from jax import lax
from jax.experimental import pallas as pl
from jax.experimental.pallas import tpu as pltpu
from jax.experimental.pallas import tpu_sc as plsc

_NE = 320000
_NN = 10000
_L = 16
_NER = _NE // 8
_BR = 2000

_C = 80
_PT = _NE // 32
_NCH = _PT // _C
_NNP = 10240
_RT = _NNP // 16


def _mlpw(p):
    return p[0]["W"], p[0]["b"], p[1]["W"], p[1]["b"]


def _enc_edges_body(x, w1, bias1, q, c, h, pre):
    hv = jnp.maximum(x[...] @ w1[...] + bias1[...], 0.0)
    h[...] = hv
    pre[...] = hv @ q[...] + c[...]


def _call_enc_edges(edges_r, w1, bias1, q, c):
    return pl.pallas_call(
        _enc_edges_body,
        grid=(_NER // _BR,),
        in_specs=[
            pl.BlockSpec((_BR, 128), lambda i: (i, 0)),
            pl.BlockSpec((128, 128), lambda i: (0, 0)),
            pl.BlockSpec((1, 128), lambda i: (0, 0)),
            pl.BlockSpec((128, 128), lambda i: (0, 0)),
            pl.BlockSpec((1, 128), lambda i: (0, 0)),
        ],
        out_specs=[pl.BlockSpec((_BR, 128), lambda i: (i, 0)),
                   pl.BlockSpec((_BR, 128), lambda i: (i, 0))],
        out_shape=[jax.ShapeDtypeStruct((_NER, 128), jnp.float32),
                   jax.ShapeDtypeStruct((_NER, 128), jnp.float32)],
    )(edges_r, w1, bias1, q, c)


def _pre_body(hp, he, pk, qk, c, out):
    out[...] = hp[...] @ pk[...] + he[...] @ qk[...] + c[...]


def _call_pre(hp_r, he_r, pk, qk, c):
    return pl.pallas_call(
        _pre_body,
        grid=(_NER // _BR,),
        in_specs=[
            pl.BlockSpec((_BR, 128), lambda i: (i, 0)),
            pl.BlockSpec((_BR, 128), lambda i: (i, 0)),
            pl.BlockSpec((128, 128), lambda i: (0, 0)),
            pl.BlockSpec((128, 128), lambda i: (0, 0)),
            pl.BlockSpec((1, 128), lambda i: (0, 0)),
        ],
        out_specs=pl.BlockSpec((_BR, 128), lambda i: (i, 0)),
        out_shape=jax.ShapeDtypeStruct((_NER, 128), jnp.float32),
    )(hp_r, he_r, pk, qk, c)


def _dec_edges_body(h2, m1, v1, m2, v2, out):
    t = jnp.maximum(h2[...] @ m1[...] + v1[...], 0.0)
    out[...] = t @ m2[...] + v2[...]


def _call_dec_edges(h2_r, m1, v1, m2, v2):
    return pl.pallas_call(
        _dec_edges_body,
        grid=(_NER // _BR,),
        in_specs=[
            pl.BlockSpec((_BR, 128), lambda i: (i, 0)),
            pl.BlockSpec((128, 128), lambda i: (0, 0)),
            pl.BlockSpec((1, 128), lambda i: (0, 0)),
            pl.BlockSpec((128, 8), lambda i: (0, 0)),
            pl.BlockSpec((1, 8), lambda i: (0, 0)),
        ],
        out_specs=pl.BlockSpec((_BR, 8), lambda i: (i, 0)),
        out_shape=jax.ShapeDtypeStruct((_NER, 8), jnp.float32),
    )(h2_r, m1, v1, m2, v2)


def _enc_nodes_body(nodes, glob, wn1, bn1, wn2, bn2, wg1, bg1, wg2, bg2,
                    s0, r0, g0w, cst0, n0, g0, ns0, nr0, c0):
    n = jnp.maximum(nodes[...] @ wn1[...] + bn1[...], 0.0) @ wn2[...] + bn2[...]
    g = jnp.maximum(glob[...] @ wg1[...] + bg1[...], 0.0) @ wg2[...] + bg2[...]
    n0[...] = n
    g0[...] = g
    ns0[...] = n @ s0[...]
    nr0[...] = n @ r0[...]
    c0[...] = g @ g0w[...] + cst0[...]


def _call_enc_nodes(nodes, glob, weights):
    return pl.pallas_call(
        _enc_nodes_body,
        out_shape=[
            jax.ShapeDtypeStruct((_NN, _L), jnp.float32),
            jax.ShapeDtypeStruct((1, _L), jnp.float32),
            jax.ShapeDtypeStruct((_NN, _L), jnp.float32),
            jax.ShapeDtypeStruct((_NN, _L), jnp.float32),
            jax.ShapeDtypeStruct((1, _L), jnp.float32),
        ],
    )(nodes, glob, *weights)


def _stage_common(hsum, cnt, n_in, g_in, w2, b2,
                  wm1, bm1, wm2, bm2, wq1, bq1, wq2, bq2):
    chat = jnp.maximum(cnt, 1.0)
    mask = (cnt > 0.0).astype(jnp.float32)
    agg = (hsum @ w2) / chat + mask * b2
    gb = jnp.broadcast_to(g_in, (_NN, g_in.shape[1]))
    x = jnp.concatenate([agg, n_in, gb], axis=1)
    n_new = jnp.maximum(x @ wm1 + bm1, 0.0) @ wm2 + bm2
    esum = jnp.sum(hsum, axis=0, keepdims=True)
    mean_e = (esum * (1.0 / _NE)) @ w2 + b2
    mean_n = jnp.sum(n_new, axis=0, keepdims=True) * (1.0 / _NN)
    gx = jnp.concatenate([mean_e, mean_n, g_in], axis=1)
    g_new = jnp.maximum(gx @ wq1 + bq1, 0.0) @ wq2 + bq2
    return n_new, g_new


def _stage01_body(is_first, *refs):
    (hsumP, cref, npv, n0, gpv, g0, w2, b2,
     wm1, bm1, wm2, bm2, wq1, bq1, wq2, bq2,
     sa, sb, ra, rb, ga, gb_, cstn,
     n_out, g_out, cnt_out, ns_out, nr_out, c_out) = refs
    hsum = (hsumP[0] + hsumP[1])[0:_NN]
    if is_first:
        cnt = (cref[0] + cref[1])[0:_NN, 0:1]
        n_in = npv[...]
        g_in = gpv[...]
    else:
        cnt = cref[...]
        n_in = jnp.concatenate([npv[...], n0[...]], axis=1)
        g_in = jnp.concatenate([gpv[...], g0[...]], axis=1)
    n_new, g_new = _stage_common(hsum, cnt, n_in, g_in, w2[...], b2[...],
                                 wm1[...], bm1[...], wm2[...], bm2[...],
                                 wq1[...], bq1[...], wq2[...], bq2[...])
    n_out[...] = n_new
    g_out[...] = g_new
    cnt_out[...] = cnt
    ns_out[...] = n_new @ sa[...] + n0[...] @ sb[...]
    nr_out[...] = n_new @ ra[...] + n0[...] @ rb[...]
    c_out[...] = g_new @ ga[...] + g0[...] @ gb_[...] + cstn[...]


def _call_stage01(is_first, args):
    return pl.pallas_call(
        functools.partial(_stage01_body, is_first),
        compiler_params=pltpu.CompilerParams(vmem_limit_bytes=100 * 1024 * 1024),
        out_shape=[
            jax.ShapeDtypeStruct((_NN, _L), jnp.float32),
            jax.ShapeDtypeStruct((1, _L), jnp.float32),
            jax.ShapeDtypeStruct((_NN, 1), jnp.float32),
            jax.ShapeDtypeStruct((_NN, _L), jnp.float32),
            jax.ShapeDtypeStruct((_NN, _L), jnp.float32),
            jax.ShapeDtypeStruct((1, _L), jnp.float32),
        ],
    )(*args)


def _stage2_body(*refs):
    (hsumP, cref, npv, n0, gpv, g0, w2, b2,
     wm1, bm1, wm2, bm2, wq1, bq1, wq2, bq2,
     dn1, dnb1, mn, vn, dg1, dgb1, mg, vg,
     nout, gout) = refs
    hsum = (hsumP[0] + hsumP[1])[0:_NN]
    cnt = cref[...]
    n_in = jnp.concatenate([npv[...], n0[...]], axis=1)
    g_in = jnp.concatenate([gpv[...], g0[...]], axis=1)
    n_new, g_new = _stage_common(hsum, cnt, n_in, g_in, w2[...], b2[...],
                                 wm1[...], bm1[...], wm2[...], bm2[...],
                                 wq1[...], bq1[...], wq2[...], bq2[...])
    tn = jnp.maximum(n_new @ dn1[...] + dnb1[...], 0.0)
    nout[...] = tn @ mn[...] + vn[...]
    tg = jnp.maximum(g_new @ dg1[...] + dgb1[...], 0.0)
    gout[...] = tg @ mg[...] + vg[...]


def _call_stage2(args):
    return pl.pallas_call(
        _stage2_body,
        compiler_params=pltpu.CompilerParams(vmem_limit_bytes=100 * 1024 * 1024),
        out_shape=[
            jax.ShapeDtypeStruct((_NN, 2), jnp.float32),
            jax.ShapeDtypeStruct((1, 1), jnp.float32),
        ],
    )(*args)


def _make_sc_pass(with_count):
    mesh = plsc.VectorSubcoreMesh(core_axis_name="c", subcore_axis_name="s")
    outs = [
        jax.ShapeDtypeStruct((_NE, _L), jnp.float32),
        jax.ShapeDtypeStruct((2, _NNP, _L), jnp.float32),
    ]
    scratch = [
        pltpu.VMEM((_C,), jnp.int32),
        pltpu.VMEM((_C,), jnp.int32),
        pltpu.VMEM((_C, _L), jnp.float32),
        pltpu.VMEM((_C, _L), jnp.float32),
        pltpu.VMEM((_C, _L), jnp.float32),
        pltpu.VMEM((_C, _L), jnp.float32),
        pltpu.VMEM((_RT, _L), jnp.float32),
        pltpu.VMEM_SHARED((_NNP, _L), jnp.float32),
        pltpu.SemaphoreType.DMA,
        pltpu.SemaphoreType.DMA,
        pltpu.SemaphoreType.DMA,
    ]
    if with_count:
        outs.append(jax.ShapeDtypeStruct((2, _NNP, _L), jnp.float32))
        scratch += [
            pltpu.VMEM((_C, _L), jnp.float32),
            pltpu.VMEM_SHARED((_NNP, _L), jnp.float32),
        ]

    def body(pre_hbm, snd_hbm, rcv_hbm, ns_hbm, nr_hbm, *rest):
        if with_count:
            (h_hbm, hsum_hbm, cnt_hbm, ids, idr, prev, srow, rrow, hv, zbuf,
             hsum_sh, sem_ld, sem1, sem2, ones_v, cnt_sh) = rest
        else:
            (h_hbm, hsum_hbm, ids, idr, prev, srow, rrow, hv, zbuf,
             hsum_sh, sem_ld, sem1, sem2) = rest
        cid = lax.axis_index("c")
        sid = lax.axis_index("s")
        wid = cid * 16 + sid

        def zrow(i, carry):
            zbuf[i] = jnp.zeros((_L,), jnp.float32)
            return carry
        lax.fori_loop(0, _RT, zrow, 0)
        pltpu.sync_copy(zbuf, hsum_sh.at[pl.ds(sid * _RT, _RT)])
        if with_count:
            pltpu.sync_copy(zbuf, cnt_sh.at[pl.ds(sid * _RT, _RT)])

            def orow(i, carry):
                ones_v[i] = jnp.ones((_L,), jnp.float32)
                return carry
            lax.fori_loop(0, _C, orow, 0)
        plsc.subcore_barrier()

        def chunk(k, carry):
            base = wid * _PT + k * _C
            l1 = pltpu.async_copy(snd_hbm.at[pl.ds(base, _C)], ids, sem_ld)
            l2 = pltpu.async_copy(rcv_hbm.at[pl.ds(base, _C)], idr, sem_ld)
            l3 = pltpu.async_copy(pre_hbm.at[pl.ds(base, _C)], prev, sem_ld)
            l1.wait()
            l2.wait()
            l3.wait()
            g1 = pltpu.async_copy(ns_hbm.at[ids], srow, sem1)
            g2 = pltpu.async_copy(nr_hbm.at[idr], rrow, sem2)
            g1.wait()
            g2.wait()

            def row(i, c2):
                hv[i] = jnp.maximum(prev[i] + srow[i] + rrow[i], 0.0)
                return c2
            lax.fori_loop(0, _C, row, 0, unroll=8)
            w1 = pltpu.async_copy(hv, h_hbm.at[pl.ds(base, _C)], sem1)
            w2 = pltpu.async_copy(hv, hsum_sh.at[idr], sem2, add=True)
            if with_count:
                w3 = pltpu.async_copy(ones_v, cnt_sh.at[idr], sem_ld, add=True)
            w1.wait()
            w2.wait()
            if with_count:
                w3.wait()
            return carry
        lax.fori_loop(0, _NCH, chunk, 0)

        plsc.subcore_barrier()
        pltpu.sync_copy(hsum_sh.at[pl.ds(sid * _RT, _RT)], zbuf)
        pltpu.sync_copy(zbuf, hsum_hbm.at[cid, pl.ds(sid * _RT, _RT)])
        if with_count:
            pltpu.sync_copy(cnt_sh.at[pl.ds(sid * _RT, _RT)], zbuf)
            pltpu.sync_copy(zbuf, cnt_hbm.at[cid, pl.ds(sid * _RT, _RT)])

    return pl.kernel(body, out_type=outs, scratch_types=scratch, mesh=mesh,
                     compiler_params=pltpu.CompilerParams(
                         use_tc_tiling_on_sc=False))


@functools.lru_cache(maxsize=2)
def _sc_pass_cached(with_count):
    return _make_sc_pass(with_count)


def _sc_pass_count(*args):
    return _sc_pass_cached(True)(*args)


def _sc_pass(*args):
    return _sc_pass_cached(False)(*args)


def kernel(nodes, edges, globals_, senders, receivers, params):
    p = params
    snd = senders.astype(jnp.int32)
    rcv = receivers.astype(jnp.int32)

    We1, be1, We2, be2 = _mlpw(p["enc"]["edge"])
    Wn1, bn1, Wn2, bn2 = _mlpw(p["enc"]["node"])
    Wg1, bg1, Wg2, bg2 = _mlpw(p["enc"]["glob"])

    W1 = [p["core%d" % i]["edge"][0]["W"] for i in range(3)]
    b1 = [p["core%d" % i]["edge"][0]["b"] for i in range(3)]
    W2 = [p["core%d" % i]["edge"][1]["W"] for i in range(3)]
    b2 = [p["core%d" % i]["edge"][1]["b"] for i in range(3)]

    E0, S0, R0, G0 = W1[0][0:16], W1[0][16:32], W1[0][32:48], W1[0][48:64]
    Ea, Eb, Sa, Sb, Ra, Rb, Ga, Gb = ({} for _ in range(8))
    for i in (1, 2):
        w = W1[i]
        Ea[i], Eb[i] = w[0:16], w[16:32]
        Sa[i], Sb[i] = w[32:48], w[48:64]
        Ra[i], Rb[i] = w[64:80], w[80:96]
        Ga[i], Gb[i] = w[96:112], w[112:128]

    eye8 = jnp.eye(8, dtype=jnp.float32)
    kron = lambda w: jnp.kron(eye8, w)
    t8 = lambda v: jnp.tile(v.reshape(1, -1), (1, 8))
    r1 = lambda v: v.reshape(1, -1)

    cst0 = r1(be2 @ E0 + b1[0])
    cstn = {i: r1(b2[i - 1] @ Ea[i] + be2 @ Eb[i] + b1[i]) for i in (1, 2)}

    enc_node_w = (Wn1, r1(bn1), Wn2, r1(bn2), Wg1, r1(bg1), Wg2, r1(bg2),
                  S0, R0, G0, cst0)
    n0, g0, ns0, nr0, c0 = _call_enc_nodes(nodes, globals_, enc_node_w)

    edges_r = edges.reshape(_NER, 128)
    henc_r, pre0_r = _call_enc_edges(edges_r, kron(We1), t8(be1),
                                     kron(We2 @ E0), jnp.tile(c0, (1, 8)))

    def node_mlp_w(i):
        wm1, bm1, wm2, bm2 = _mlpw(p["core%d" % i]["node"])
        wq1, bq1, wq2, bq2 = _mlpw(p["core%d" % i]["glob"])
        return (wm1, r1(bm1), wm2, r1(bm2), wq1, r1(bq1), wq2, r1(bq2))

    h0, hsumP0, cntP0 = _sc_pass_count(pre0_r.reshape(_NE, _L), snd, rcv,
                                       ns0, nr0)
    args0 = (hsumP0, cntP0, n0, n0, g0, g0, W2[0], r1(b2[0]),
             *node_mlp_w(0), Sa[1], Sb[1], Ra[1], Rb[1], Ga[1], Gb[1], cstn[1])
    n1, g1, cnt, ns1, nr1, c1 = _call_stage01(True, args0)

    pre1_r = _call_pre(h0.reshape(_NER, 128), henc_r,
                       kron(W2[0] @ Ea[1]), kron(We2 @ Eb[1]),
                       jnp.tile(c1, (1, 8)))
    h1, hsumP1 = _sc_pass(pre1_r.reshape(_NE, _L), snd, rcv, ns1, nr1)
    args1 = (hsumP1, cnt, n1, n0, g1, g0, W2[1], r1(b2[1]),
             *node_mlp_w(1), Sa[2], Sb[2], Ra[2], Rb[2], Ga[2], Gb[2], cstn[2])
    n2, g2, _, ns2, nr2, c2 = _call_stage01(False, args1)

    pre2_r = _call_pre(h1.reshape(_NER, 128), henc_r,
                       kron(W2[1] @ Ea[2]), kron(We2 @ Eb[2]),
                       jnp.tile(c2, (1, 8)))
    h2, hsumP2 = _sc_pass(pre2_r.reshape(_NE, _L), snd, rcv, ns2, nr2)

    De1, de1, De2, de2 = _mlpw(p["dec"]["edge"])
    Dn1, dn1, Dn2, dn2 = _mlpw(p["dec"]["node"])
    Dg1, dg1, Dg2, dg2 = _mlpw(p["dec"]["glob"])
    Oe, oe = p["out"]["edge"]["W"], p["out"]["edge"]["b"]
    On, on = p["out"]["node"]["W"], p["out"]["node"]["b"]
    Og, og = p["out"]["glob"]["W"], p["out"]["glob"]["b"]

    args2 = (hsumP2, cnt, n2, n0, g2, g0, W2[2], r1(b2[2]), *node_mlp_w(2),
             Dn1, r1(dn1), Dn2 @ On, r1(dn2 @ On + on),
             Dg1, r1(dg1), Dg2 @ Og, r1(dg2 @ Og + og))
    out_n, out_g = _call_stage2(args2)

    m2e = De2 @ Oe
    v2e = (de2 @ Oe + oe)
    oute_r = _call_dec_edges(h2.reshape(_NER, 128),
                             kron(W2[2] @ De1), t8(b2[2] @ De1 + de1),
                             kron(m2e), t8(v2e))
    out_e = oute_r.reshape(_NE, 1)

    return out_n, out_e, out_g


# speedup vs baseline: 7.5346x; 1.1785x over previous
import functools

import jax
import jax.numpy as jnp
from jax import lax
from jax.experimental import pallas as pl
from jax.experimental.pallas import tpu as pltpu
from jax.experimental.pallas import tpu_sc as plsc

_NE = 320000
_NN = 10000
_L = 16
_NER = _NE // 8
_BR = 2000

_C = 80
_PT = _NE // 32
_NCH = _PT // _C
_NNP = 10240
_RT = _NNP // 16


def _mlpw(p):
    return p[0]["W"], p[0]["b"], p[1]["W"], p[1]["b"]


def _enc_edges_body(x, w1, bias1, w2, bias2, e0k, c, e0, pre):
    hv = jnp.maximum(x[...] @ w1[...] + bias1[...], 0.0)
    ev = hv @ w2[...] + bias2[...]
    e0[...] = ev
    pre[...] = ev @ e0k[...] + c[...]


def _call_enc_edges(edges_r, w1, bias1, w2, bias2, e0k, c):
    return pl.pallas_call(
        _enc_edges_body,
        grid=(_NER // _BR,),
        in_specs=[
            pl.BlockSpec((_BR, 128), lambda i: (i, 0)),
            pl.BlockSpec((128, 128), lambda i: (0, 0)),
            pl.BlockSpec((1, 128), lambda i: (0, 0)),
            pl.BlockSpec((128, 128), lambda i: (0, 0)),
            pl.BlockSpec((1, 128), lambda i: (0, 0)),
            pl.BlockSpec((128, 128), lambda i: (0, 0)),
            pl.BlockSpec((1, 128), lambda i: (0, 0)),
        ],
        out_specs=[pl.BlockSpec((_BR, 128), lambda i: (i, 0)),
                   pl.BlockSpec((_BR, 128), lambda i: (i, 0))],
        out_shape=[jax.ShapeDtypeStruct((_NER, 128), jnp.float32),
                   jax.ShapeDtypeStruct((_NER, 128), jnp.float32)],
    )(edges_r, w1, bias1, w2, bias2, e0k, c)


def _pre_body(hp, e0, w2k, b2t, eak, ebk, c, out):
    ev = hp[...] @ w2k[...] + b2t[...]
    out[...] = ev @ eak[...] + e0[...] @ ebk[...] + c[...]


def _call_pre(hp_r, e0_r, w2k, b2t, eak, ebk, c):
    return pl.pallas_call(
        _pre_body,
        grid=(_NER // _BR,),
        in_specs=[
            pl.BlockSpec((_BR, 128), lambda i: (i, 0)),
            pl.BlockSpec((_BR, 128), lambda i: (i, 0)),
            pl.BlockSpec((128, 128), lambda i: (0, 0)),
            pl.BlockSpec((1, 128), lambda i: (0, 0)),
            pl.BlockSpec((128, 128), lambda i: (0, 0)),
            pl.BlockSpec((128, 128), lambda i: (0, 0)),
            pl.BlockSpec((1, 128), lambda i: (0, 0)),
        ],
        out_specs=pl.BlockSpec((_BR, 128), lambda i: (i, 0)),
        out_shape=jax.ShapeDtypeStruct((_NER, 128), jnp.float32),
    )(hp_r, e0_r, w2k, b2t, eak, ebk, c)


def _dec_edges_body(h2, w2k, b2t, d1k, d1t, d2k, d2t, ok, ot, out):
    ev = h2[...] @ w2k[...] + b2t[...]
    t = jnp.maximum(ev @ d1k[...] + d1t[...], 0.0)
    u = t @ d2k[...] + d2t[...]
    out[...] = u @ ok[...] + ot[...]


def _call_dec_edges(h2_r, w2k, b2t, d1k, d1t, d2k, d2t, ok, ot):
    return pl.pallas_call(
        _dec_edges_body,
        grid=(_NER // _BR,),
        in_specs=[
            pl.BlockSpec((_BR, 128), lambda i: (i, 0)),
            pl.BlockSpec((128, 128), lambda i: (0, 0)),
            pl.BlockSpec((1, 128), lambda i: (0, 0)),
            pl.BlockSpec((128, 128), lambda i: (0, 0)),
            pl.BlockSpec((1, 128), lambda i: (0, 0)),
            pl.BlockSpec((128, 128), lambda i: (0, 0)),
            pl.BlockSpec((1, 128), lambda i: (0, 0)),
            pl.BlockSpec((128, 8), lambda i: (0, 0)),
            pl.BlockSpec((1, 8), lambda i: (0, 0)),
        ],
        out_specs=pl.BlockSpec((_BR, 8), lambda i: (i, 0)),
        out_shape=jax.ShapeDtypeStruct((_NER, 8), jnp.float32),
    )(h2_r, w2k, b2t, d1k, d1t, d2k, d2t, ok, ot)


def _enc_nodes_body(nodes, glob, wn1, bn1, wn2, bn2, wg1, bg1, wg2, bg2,
                    s0, r0, g0w, cst0, n0, g0, ns0, nr0, c0):
    n = jnp.maximum(nodes[...] @ wn1[...] + bn1[...], 0.0) @ wn2[...] + bn2[...]
    g = jnp.maximum(glob[...] @ wg1[...] + bg1[...], 0.0) @ wg2[...] + bg2[...]
    n0[...] = n
    g0[...] = g
    ns0[...] = n @ s0[...]
    nr0[...] = n @ r0[...]
    c0[...] = g @ g0w[...] + cst0[...]


def _call_enc_nodes(nodes, glob, weights):
    return pl.pallas_call(
        _enc_nodes_body,
        out_shape=[
            jax.ShapeDtypeStruct((_NN, _L), jnp.float32),
            jax.ShapeDtypeStruct((1, _L), jnp.float32),
            jax.ShapeDtypeStruct((_NN, _L), jnp.float32),
            jax.ShapeDtypeStruct((_NN, _L), jnp.float32),
            jax.ShapeDtypeStruct((1, _L), jnp.float32),
        ],
    )(nodes, glob, *weights)


def _stage_common(hsum, cnt, n_in, g_in, w2, b2,
                  wm1, bm1, wm2, bm2, wq1, bq1, wq2, bq2):
    chat = jnp.maximum(cnt, 1.0)
    mask = (cnt > 0.0).astype(jnp.float32)
    agg = (hsum @ w2) / chat + mask * b2
    gb = jnp.broadcast_to(g_in, (_NN, g_in.shape[1]))
    x = jnp.concatenate([agg, n_in, gb], axis=1)
    n_new = jnp.maximum(x @ wm1 + bm1, 0.0) @ wm2 + bm2
    esum = jnp.sum(hsum, axis=0, keepdims=True)
    mean_e = (esum * (1.0 / _NE)) @ w2 + b2
    mean_n = jnp.sum(n_new, axis=0, keepdims=True) * (1.0 / _NN)
    gx = jnp.concatenate([mean_e, mean_n, g_in], axis=1)
    g_new = jnp.maximum(gx @ wq1 + bq1, 0.0) @ wq2 + bq2
    return n_new, g_new


def _stage01_body(is_first, *refs):
    (hsumP, cref, npv, n0, gpv, g0, w2, b2,
     wm1, bm1, wm2, bm2, wq1, bq1, wq2, bq2,
     sa, sb, ra, rb, ga, gb_, cstn,
     n_out, g_out, cnt_out, ns_out, nr_out, c_out) = refs
    hsum = (hsumP[0] + hsumP[1])[0:_NN]
    if is_first:
        cnt = (cref[0] + cref[1])[0:_NN, 0:1]
        n_in = npv[...]
        g_in = gpv[...]
    else:
        cnt = cref[...]
        n_in = jnp.concatenate([npv[...], n0[...]], axis=1)
        g_in = jnp.concatenate([gpv[...], g0[...]], axis=1)
    n_new, g_new = _stage_common(hsum, cnt, n_in, g_in, w2[...], b2[...],
                                 wm1[...], bm1[...], wm2[...], bm2[...],
                                 wq1[...], bq1[...], wq2[...], bq2[...])
    n_out[...] = n_new
    g_out[...] = g_new
    cnt_out[...] = cnt
    ns_out[...] = n_new @ sa[...] + n0[...] @ sb[...]
    nr_out[...] = n_new @ ra[...] + n0[...] @ rb[...]
    c_out[...] = g_new @ ga[...] + g0[...] @ gb_[...] + cstn[...]


def _call_stage01(is_first, args):
    return pl.pallas_call(
        functools.partial(_stage01_body, is_first),
        compiler_params=pltpu.CompilerParams(vmem_limit_bytes=100 * 1024 * 1024),
        out_shape=[
            jax.ShapeDtypeStruct((_NN, _L), jnp.float32),
            jax.ShapeDtypeStruct((1, _L), jnp.float32),
            jax.ShapeDtypeStruct((_NN, 1), jnp.float32),
            jax.ShapeDtypeStruct((_NN, _L), jnp.float32),
            jax.ShapeDtypeStruct((_NN, _L), jnp.float32),
            jax.ShapeDtypeStruct((1, _L), jnp.float32),
        ],
    )(*args)


def _stage2_body(*refs):
    (hsumP, cref, npv, n0, gpv, g0, w2, b2,
     wm1, bm1, wm2, bm2, wq1, bq1, wq2, bq2,
     dn1, dnb1, dn2, dnb2, onw, onb, dg1, dgb1, dg2, dgb2, ogw, ogb,
     nout, gout) = refs
    hsum = (hsumP[0] + hsumP[1])[0:_NN]
    cnt = cref[...]
    n_in = jnp.concatenate([npv[...], n0[...]], axis=1)
    g_in = jnp.concatenate([gpv[...], g0[...]], axis=1)
    n_new, g_new = _stage_common(hsum, cnt, n_in, g_in, w2[...], b2[...],
                                 wm1[...], bm1[...], wm2[...], bm2[...],
                                 wq1[...], bq1[...], wq2[...], bq2[...])
    tn = jnp.maximum(n_new @ dn1[...] + dnb1[...], 0.0)
    un = tn @ dn2[...] + dnb2[...]
    nout[...] = un @ onw[...] + onb[...]
    tg = jnp.maximum(g_new @ dg1[...] + dgb1[...], 0.0)
    ug = tg @ dg2[...] + dgb2[...]
    gout[...] = ug @ ogw[...] + ogb[...]


def _call_stage2(args):
    return pl.pallas_call(
        _stage2_body,
        compiler_params=pltpu.CompilerParams(vmem_limit_bytes=100 * 1024 * 1024),
        out_shape=[
            jax.ShapeDtypeStruct((_NN, 2), jnp.float32),
            jax.ShapeDtypeStruct((1, 1), jnp.float32),
        ],
    )(*args)


def _make_sc_pass(with_count):
    mesh = plsc.VectorSubcoreMesh(core_axis_name="c", subcore_axis_name="s")
    outs = [
        jax.ShapeDtypeStruct((_NE, _L), jnp.float32),
        jax.ShapeDtypeStruct((2, _NNP, _L), jnp.float32),
    ]
    scratch = [
        pltpu.VMEM((_C,), jnp.int32),
        pltpu.VMEM((_C,), jnp.int32),
        pltpu.VMEM((_C, _L), jnp.float32),
        pltpu.VMEM((_C, _L), jnp.float32),
        pltpu.VMEM((_C, _L), jnp.float32),
        pltpu.VMEM((_C, _L), jnp.float32),
        pltpu.VMEM((_RT, _L), jnp.float32),
        pltpu.VMEM_SHARED((_NNP, _L), jnp.float32),
        pltpu.VMEM_SHARED((_NN, _L), jnp.float32),
        pltpu.VMEM_SHARED((_NN, _L), jnp.float32),
        pltpu.SemaphoreType.DMA,
        pltpu.SemaphoreType.DMA,
        pltpu.SemaphoreType.DMA,
    ]
    if with_count:
        outs.append(jax.ShapeDtypeStruct((2, _NNP, _L), jnp.float32))
        scratch += [
            pltpu.VMEM((_C, _L), jnp.float32),
            pltpu.VMEM_SHARED((_NNP, _L), jnp.float32),
        ]

    def body(pre_hbm, snd_hbm, rcv_hbm, ns_hbm, nr_hbm, *rest):
        if with_count:
            (h_hbm, hsum_hbm, cnt_hbm, ids, idr, prev, srow, rrow, hv, zbuf,
             hsum_sh, ns_sh, nr_sh, sem_ld, sem1, sem2, ones_v, cnt_sh) = rest
        else:
            (h_hbm, hsum_hbm, ids, idr, prev, srow, rrow, hv, zbuf,
             hsum_sh, ns_sh, nr_sh, sem_ld, sem1, sem2) = rest
        cid = lax.axis_index("c")
        sid = lax.axis_index("s")
        wid = cid * 16 + sid

        def zrow(i, carry):
            zbuf[i] = jnp.zeros((_L,), jnp.float32)
            return carry
        lax.fori_loop(0, _RT, zrow, 0)
        pltpu.sync_copy(zbuf, hsum_sh.at[pl.ds(sid * _RT, _RT)])
        if with_count:
            pltpu.sync_copy(zbuf, cnt_sh.at[pl.ds(sid * _RT, _RT)])

            def orow(i, carry):
                ones_v[i] = jnp.ones((_L,), jnp.float32)
                return carry
            lax.fori_loop(0, _C, orow, 0)
        slab = _NN // 16
        pltpu.sync_copy(ns_hbm.at[pl.ds(sid * slab, slab)],
                        zbuf.at[pl.ds(0, slab)])
        pltpu.sync_copy(zbuf.at[pl.ds(0, slab)],
                        ns_sh.at[pl.ds(sid * slab, slab)])
        pltpu.sync_copy(nr_hbm.at[pl.ds(sid * slab, slab)],
                        zbuf.at[pl.ds(0, slab)])
        pltpu.sync_copy(zbuf.at[pl.ds(0, slab)],
                        nr_sh.at[pl.ds(sid * slab, slab)])
        plsc.subcore_barrier()

        def chunk(k, carry):
            base = wid * _PT + k * _C
            l1 = pltpu.async_copy(snd_hbm.at[pl.ds(base, _C)], ids, sem_ld)
            l2 = pltpu.async_copy(rcv_hbm.at[pl.ds(base, _C)], idr, sem_ld)
            l3 = pltpu.async_copy(pre_hbm.at[pl.ds(base, _C)], prev, sem_ld)
            l1.wait()
            l2.wait()
            l3.wait()
            g1 = pltpu.async_copy(ns_sh.at[ids], srow, sem1)
            g2 = pltpu.async_copy(nr_sh.at[idr], rrow, sem2)
            g1.wait()
            g2.wait()

            def row(i, c2):
                hv[i] = jnp.maximum(prev[i] + srow[i] + rrow[i], 0.0)
                return c2
            lax.fori_loop(0, _C, row, 0, unroll=8)
            w1 = pltpu.async_copy(hv, h_hbm.at[pl.ds(base, _C)], sem1)
            w2 = pltpu.async_copy(hv, hsum_sh.at[idr], sem2, add=True)
            if with_count:
                w3 = pltpu.async_copy(ones_v, cnt_sh.at[idr], sem_ld, add=True)
            w1.wait()
            w2.wait()
            if with_count:
                w3.wait()
            return carry
        lax.fori_loop(0, _NCH, chunk, 0)

        plsc.subcore_barrier()
        pltpu.sync_copy(hsum_sh.at[pl.ds(sid * _RT, _RT)], zbuf)
        pltpu.sync_copy(zbuf, hsum_hbm.at[cid, pl.ds(sid * _RT, _RT)])
        if with_count:
            pltpu.sync_copy(cnt_sh.at[pl.ds(sid * _RT, _RT)], zbuf)
            pltpu.sync_copy(zbuf, cnt_hbm.at[cid, pl.ds(sid * _RT, _RT)])

    return pl.kernel(body, out_type=outs, scratch_types=scratch, mesh=mesh,
                     compiler_params=pltpu.CompilerParams(
                         use_tc_tiling_on_sc=False))


@functools.lru_cache(maxsize=2)
def _sc_pass_cached(with_count):
    return _make_sc_pass(with_count)


def _sc_pass_count(*args):
    return _sc_pass_cached(True)(*args)


def _sc_pass(*args):
    return _sc_pass_cached(False)(*args)


def kernel(nodes, edges, globals_, senders, receivers, params):
    p = params
    snd = senders.astype(jnp.int32)
    rcv = receivers.astype(jnp.int32)

    We1, be1, We2, be2 = _mlpw(p["enc"]["edge"])
    Wn1, bn1, Wn2, bn2 = _mlpw(p["enc"]["node"])
    Wg1, bg1, Wg2, bg2 = _mlpw(p["enc"]["glob"])

    W1 = [p["core%d" % i]["edge"][0]["W"] for i in range(3)]
    b1 = [p["core%d" % i]["edge"][0]["b"] for i in range(3)]
    W2 = [p["core%d" % i]["edge"][1]["W"] for i in range(3)]
    b2 = [p["core%d" % i]["edge"][1]["b"] for i in range(3)]

    E0, S0, R0, G0 = W1[0][0:16], W1[0][16:32], W1[0][32:48], W1[0][48:64]
    Ea, Eb, Sa, Sb, Ra, Rb, Ga, Gb = ({} for _ in range(8))
    for i in (1, 2):
        w = W1[i]
        Ea[i], Eb[i] = w[0:16], w[16:32]
        Sa[i], Sb[i] = w[32:48], w[48:64]
        Ra[i], Rb[i] = w[64:80], w[80:96]
        Ga[i], Gb[i] = w[96:112], w[112:128]

    eye8 = jnp.eye(8, dtype=jnp.float32)
    kron = lambda w: jnp.kron(eye8, w)
    t8 = lambda v: jnp.tile(v.reshape(1, -1), (1, 8))
    r1 = lambda v: v.reshape(1, -1)

    cst0 = r1(be2 @ E0 + b1[0])
    cstn = {i: r1(b2[i - 1] @ Ea[i] + be2 @ Eb[i] + b1[i]) for i in (1, 2)}

    enc_node_w = (Wn1, r1(bn1), Wn2, r1(bn2), Wg1, r1(bg1), Wg2, r1(bg2),
                  S0, R0, G0, cst0)
    n0, g0, ns0, nr0, c0 = _call_enc_nodes(nodes, globals_, enc_node_w)

    edges_r = edges.reshape(_NER, 128)
    e0_r, pre0_r = _call_enc_edges(edges_r, kron(We1), t8(be1),
                                   kron(We2), t8(be2),
                                   kron(E0), jnp.tile(c0, (1, 8)))

    def node_mlp_w(i):
        wm1, bm1, wm2, bm2 = _mlpw(p["core%d" % i]["node"])
        wq1, bq1, wq2, bq2 = _mlpw(p["core%d" % i]["glob"])
        return (wm1, r1(bm1), wm2, r1(bm2), wq1, r1(bq1), wq2, r1(bq2))

    h0, hsumP0, cntP0 = _sc_pass_count(pre0_r.reshape(_NE, _L), snd, rcv,
                                       ns0, nr0)
    args0 = (hsumP0, cntP0, n0, n0, g0, g0, W2[0], r1(b2[0]),
             *node_mlp_w(0), Sa[1], Sb[1], Ra[1], Rb[1], Ga[1], Gb[1], cstn[1])
    n1, g1, cnt, ns1, nr1, c1 = _call_stage01(True, args0)

    pre1_r = _call_pre(h0.reshape(_NER, 128), e0_r,
                       kron(W2[0]), t8(b2[0]), kron(Ea[1]), kron(Eb[1]),
                       jnp.tile(c1, (1, 8)))
    h1, hsumP1 = _sc_pass(pre1_r.reshape(_NE, _L), snd, rcv, ns1, nr1)
    args1 = (hsumP1, cnt, n1, n0, g1, g0, W2[1], r1(b2[1]),
             *node_mlp_w(1), Sa[2], Sb[2], Ra[2], Rb[2], Ga[2], Gb[2], cstn[2])
    n2, g2, _, ns2, nr2, c2 = _call_stage01(False, args1)

    pre2_r = _call_pre(h1.reshape(_NER, 128), e0_r,
                       kron(W2[1]), t8(b2[1]), kron(Ea[2]), kron(Eb[2]),
                       jnp.tile(c2, (1, 8)))
    h2, hsumP2 = _sc_pass(pre2_r.reshape(_NE, _L), snd, rcv, ns2, nr2)

    De1, de1, De2, de2 = _mlpw(p["dec"]["edge"])
    Dn1, dn1, Dn2, dn2 = _mlpw(p["dec"]["node"])
    Dg1, dg1, Dg2, dg2 = _mlpw(p["dec"]["glob"])
    Oe, oe = p["out"]["edge"]["W"], p["out"]["edge"]["b"]
    On, on = p["out"]["node"]["W"], p["out"]["node"]["b"]
    Og, og = p["out"]["glob"]["W"], p["out"]["glob"]["b"]

    args2 = (hsumP2, cnt, n2, n0, g2, g0, W2[2], r1(b2[2]), *node_mlp_w(2),
             Dn1, r1(dn1), Dn2, r1(dn2), On, r1(on),
             Dg1, r1(dg1), Dg2, r1(dg2), Og, r1(og))
    out_n, out_g = _call_stage2(args2)

    oute_r = _call_dec_edges(h2.reshape(_NER, 128),
                             kron(W2[2]), t8(b2[2]),
                             kron(De1), t8(de1), kron(De2), t8(de2),
                             kron(Oe), t8(oe))
    out_e = oute_r.reshape(_NE, 1)

    return out_n, out_e, out_g
